# baseline (speedup 1.0000x reference)
import jax
import jax.numpy as jnp
from jax import lax
from jax.experimental import pallas as pl
from jax.experimental.pallas import tpu as pltpu, tpu_sc as plsc

MASK = 103.0

B = 8192
D = 2048
NC = 2
NS = 16
NW = NC * NS
RPW = B // NW
CHUNK = 8
NCHUNK = RPW // CHUNK
NBUF = 6
L = 16


def _body(x_hbm, idx_hbm, out_hbm, idx_v, *rest):
    bufs = rest[:NBUF]
    isems = rest[NBUF:2 * NBUF]
    osems = rest[2 * NBUF:3 * NBUF]

    wid = lax.axis_index("s") * NC + lax.axis_index("c")
    base = wid * RPW

    in_dma = [None] * NBUF
    out_dma = [None] * NBUF

    def gather(c, b):
        return pltpu.async_copy(
            x_hbm.at[pl.ds(base + c * CHUNK, CHUNK), :], bufs[b], isems[b])

    def scatter(c, b):
        return pltpu.async_copy(
            bufs[b], out_hbm.at[pl.ds(base + c * CHUNK, CHUNK), :], osems[b])

    for c in range(min(NBUF, NCHUNK)):
        in_dma[c] = gather(c, c)

    pltpu.sync_copy(idx_hbm.at[pl.ds(base, RPW)], idx_v.at[pl.ds(0, RPW)])

    rows = lax.iota(jnp.int32, L)
    lane_ok = rows < CHUNK
    vals = jnp.full((L,), MASK, dtype=jnp.float32)

    for c in range(NCHUNK):
        b = c % NBUF
        n = c + 1
        if NBUF <= n < NCHUNK:
            nb = n % NBUF
            out_dma[nb].wait()
            in_dma[nb] = gather(n, nb)
        in_dma[b].wait()
        cols = idx_v[pl.ds(c * CHUNK, L)]
        plsc.store_scatter(bufs[b], [rows, cols], vals, mask=lane_ok)
        out_dma[b] = scatter(c, b)

    for k in range(min(NBUF, NCHUNK)):
        out_dma[(NCHUNK - 1 - k) % NBUF].wait()


_sc_mask = pl.kernel(
    _body,
    out_type=jax.ShapeDtypeStruct((B, D), jnp.float32),
    mesh=plsc.VectorSubcoreMesh(core_axis_name="c", subcore_axis_name="s"),
    compiler_params=pltpu.CompilerParams(needs_layout_passes=False),
    scratch_types=(
        [pltpu.VMEM((RPW + L,), jnp.int32)]
        + [pltpu.VMEM((CHUNK, D), jnp.float32) for _ in range(NBUF)]
        + [pltpu.SemaphoreType.DMA for _ in range(2 * NBUF)]
    ),
)


@jax.jit
def kernel(x, idx):
    cols = idx.reshape(B).astype(jnp.int32)
    return _sc_mask(x, cols)


# speedup vs baseline: 1.0019x; 1.0019x over previous
import jax
import jax.numpy as jnp
from jax import lax
from jax.experimental import pallas as pl
from jax.experimental.pallas import tpu as pltpu, tpu_sc as plsc

MASK = 103.0

B = 8192
D = 2048
NC = 2
NS = 16
NW = NC * NS
RPW = B // NW
CHUNK = 8
NCHUNK = RPW // CHUNK
NBUF = 7
L = 16


def _body(x_hbm, idx_hbm, out_hbm, idx_v, *rest):
    bufs = rest[:NBUF]
    isems = rest[NBUF:2 * NBUF]
    osems = rest[2 * NBUF:3 * NBUF]

    wid = lax.axis_index("s") * NC + lax.axis_index("c")
    base = wid * RPW

    in_dma = [None] * NBUF
    out_dma = [None] * NBUF

    def gather(c, b):
        return pltpu.async_copy(
            x_hbm.at[pl.ds(base + c * CHUNK, CHUNK), :], bufs[b], isems[b])

    def scatter(c, b):
        return pltpu.async_copy(
            bufs[b], out_hbm.at[pl.ds(base + c * CHUNK, CHUNK), :], osems[b])

    for c in range(min(NBUF, NCHUNK)):
        in_dma[c] = gather(c, c)

    pltpu.sync_copy(idx_hbm.at[pl.ds(base, RPW)], idx_v.at[pl.ds(0, RPW)])

    rows = lax.iota(jnp.int32, L)
    lane_ok = rows < CHUNK
    vals = jnp.full((L,), MASK, dtype=jnp.float32)

    for c in range(NCHUNK):
        b = c % NBUF
        n = c + 1
        if NBUF <= n < NCHUNK:
            nb = n % NBUF
            out_dma[nb].wait()
            in_dma[nb] = gather(n, nb)
        in_dma[b].wait()
        cols = idx_v[pl.ds(c * CHUNK, L)]
        plsc.store_scatter(bufs[b], [rows, cols], vals, mask=lane_ok)
        out_dma[b] = scatter(c, b)

    for k in range(min(NBUF, NCHUNK)):
        out_dma[(NCHUNK - 1 - k) % NBUF].wait()


_sc_mask = pl.kernel(
    _body,
    out_type=jax.ShapeDtypeStruct((B, D), jnp.float32),
    mesh=plsc.VectorSubcoreMesh(core_axis_name="c", subcore_axis_name="s"),
    compiler_params=pltpu.CompilerParams(needs_layout_passes=False),
    scratch_types=(
        [pltpu.VMEM((RPW + L,), jnp.int32)]
        + [pltpu.VMEM((CHUNK, D), jnp.float32) for _ in range(NBUF)]
        + [pltpu.SemaphoreType.DMA for _ in range(2 * NBUF)]
    ),
)


@jax.jit
def kernel(x, idx):
    cols = idx.reshape(B).astype(jnp.int32)
    return _sc_mask(x, cols)


# speedup vs baseline: 1.0023x; 1.0004x over previous
import jax
import jax.numpy as jnp
from jax import lax
from jax.experimental import pallas as pl
from jax.experimental.pallas import tpu as pltpu, tpu_sc as plsc

MASK = 103.0

B = 8192
D = 2048
NC = 2
NS = 16
NW = NC * NS
RPW = B // NW
CHUNK = 8
NCHUNK = RPW // CHUNK
NBUF = 7
L = 16


def _body(x_hbm, idx_hbm, out_hbm, idx_v, *rest):
    bufs = rest[:NBUF]
    isems = rest[NBUF:2 * NBUF]
    osems = rest[2 * NBUF:3 * NBUF]

    wid = lax.axis_index("s") * NC + lax.axis_index("c")
    base = wid * RPW

    in_dma = [None] * NBUF
    out_dma = [None] * NBUF

    def gather(c, b):
        return pltpu.async_copy(
            x_hbm.at[pl.ds(base + c * CHUNK, CHUNK), :], bufs[b], isems[b])

    def scatter(c, b):
        return pltpu.async_copy(
            bufs[b], out_hbm.at[pl.ds(base + c * CHUNK, CHUNK), :], osems[b])

    for c in range(min(NBUF, NCHUNK)):
        in_dma[c] = gather(c, c)

    pltpu.sync_copy(idx_hbm.at[pl.ds(base, RPW)], idx_v.at[pl.ds(0, RPW)])
    idx_v[pl.ds(RPW, L)] = jnp.zeros((L,), dtype=jnp.int32)

    rows = lax.iota(jnp.int32, L)
    lane_ok = rows < CHUNK
    vals = jnp.full((L,), MASK, dtype=jnp.float32)

    for c in range(NCHUNK):
        b = c % NBUF
        n = c + 1
        if NBUF <= n < NCHUNK:
            nb = n % NBUF
            out_dma[nb].wait()
            in_dma[nb] = gather(n, nb)
        in_dma[b].wait()
        cols = idx_v[pl.ds(c * CHUNK, L)]
        plsc.store_scatter(bufs[b], [rows, cols], vals, mask=lane_ok)
        out_dma[b] = scatter(c, b)

    for k in range(min(NBUF, NCHUNK)):
        out_dma[(NCHUNK - 1 - k) % NBUF].wait()


_sc_mask = pl.kernel(
    _body,
    out_type=jax.ShapeDtypeStruct((B, D), jnp.float32),
    mesh=plsc.VectorSubcoreMesh(core_axis_name="c", subcore_axis_name="s"),
    compiler_params=pltpu.CompilerParams(needs_layout_passes=False),
    scratch_types=(
        [pltpu.VMEM((RPW + L,), jnp.int32)]
        + [pltpu.VMEM((CHUNK, D), jnp.float32) for _ in range(NBUF)]
        + [pltpu.SemaphoreType.DMA for _ in range(2 * NBUF)]
    ),
)


@jax.jit
def kernel(x, idx):
    cols = idx.reshape(B).astype(jnp.int32)
    return _sc_mask(x, cols)
